# baseline (speedup 1.0000x reference)
import jax
import jax.numpy as jnp
from jax.experimental import pallas as pl

_BT = 4096


def _embed_kernel(x_ref, w0, b0, w1, b1, w2, b2, w3, b3, o_ref):
    ws = [w0, w1, w2, w3]
    bs = [b0, b1, b2, b3]
    wb = jnp.concatenate(
        [
            jnp.pad(ws[i][...], ((0, 0), (32 * i, 96 - 32 * i)))
            for i in range(4)
        ],
        axis=0,
    )
    bb = jnp.concatenate([b[...] for b in bs], axis=1)
    o_ref[...] = (
        jnp.dot(x_ref[...], wb, preferred_element_type=jnp.float32) + bb
    )


@jax.jit
def kernel(x, W0, b0, W1, b1, W2, b2, W3, b3):
    G, H = W0.shape
    D = G * 4
    O = H * 4
    B = x.shape[0]
    wspec = pl.BlockSpec((G, H), lambda i: (0, 0))
    bspec = pl.BlockSpec((1, H), lambda i: (0, 0))
    return pl.pallas_call(
        _embed_kernel,
        grid=(B // _BT,),
        in_specs=[
            pl.BlockSpec((_BT, D), lambda i: (i, 0)),
            wspec, bspec, wspec, bspec, wspec, bspec, wspec, bspec,
        ],
        out_specs=pl.BlockSpec((_BT, O), lambda i: (i, 0)),
        out_shape=jax.ShapeDtypeStruct((B, O), x.dtype),
    )(
        x,
        W0, b0.reshape(1, H),
        W1, b1.reshape(1, H),
        W2, b2.reshape(1, H),
        W3, b3.reshape(1, H),
    )


# speedup vs baseline: 1.0157x; 1.0157x over previous
import jax
import jax.numpy as jnp
from jax.experimental import pallas as pl
from jax.experimental.pallas import tpu as pltpu

_CHUNKS = (8192, 4096, 2048, 2048)
_NC = len(_CHUNKS)
_OFFS = tuple(sum(_CHUNKS[:i]) for i in range(_NC))


def _embed_kernel(x_hbm, w0, b0, w1, b1, w2, b2, w3, b3, o_hbm,
                  x_vmem, o_vmem, in_sems, out_sems):
    in_copies = []
    for i in range(_NC):
        c = pltpu.make_async_copy(
            x_hbm.at[pl.ds(_OFFS[i], _CHUNKS[i]), :],
            x_vmem.at[pl.ds(_OFFS[i], _CHUNKS[i]), :],
            in_sems.at[i],
        )
        c.start()
        in_copies.append(c)

    ws = [w0, w1, w2, w3]
    bs = [b0, b1, b2, b3]
    wb = jnp.concatenate(
        [
            jnp.pad(ws[i][...], ((0, 0), (32 * i, 96 - 32 * i)))
            for i in range(4)
        ],
        axis=0,
    )
    bb = jnp.concatenate([b[...] for b in bs], axis=1)

    out_copies = []
    for i in range(_NC):
        in_copies[i].wait()
        o_vmem[pl.ds(_OFFS[i], _CHUNKS[i]), :] = (
            jnp.dot(
                x_vmem[pl.ds(_OFFS[i], _CHUNKS[i]), :],
                wb,
                preferred_element_type=jnp.float32,
            )
            + bb
        )
        c = pltpu.make_async_copy(
            o_vmem.at[pl.ds(_OFFS[i], _CHUNKS[i]), :],
            o_hbm.at[pl.ds(_OFFS[i], _CHUNKS[i]), :],
            out_sems.at[i],
        )
        c.start()
        out_copies.append(c)
    for c in out_copies:
        c.wait()


@jax.jit
def kernel(x, W0, b0, W1, b1, W2, b2, W3, b3):
    G, H = W0.shape
    D = G * 4
    O = H * 4
    B = x.shape[0]
    vspec = pl.BlockSpec(memory_space=pltpu.VMEM)
    return pl.pallas_call(
        _embed_kernel,
        in_specs=[pl.BlockSpec(memory_space=pltpu.MemorySpace.HBM)]
        + [vspec] * 8,
        out_specs=pl.BlockSpec(memory_space=pltpu.MemorySpace.HBM),
        out_shape=jax.ShapeDtypeStruct((B, O), x.dtype),
        scratch_shapes=[
            pltpu.VMEM((B, D), x.dtype),
            pltpu.VMEM((B, O), x.dtype),
            pltpu.SemaphoreType.DMA((_NC,)),
            pltpu.SemaphoreType.DMA((_NC,)),
        ],
    )(
        x,
        W0, b0.reshape(1, H),
        W1, b1.reshape(1, H),
        W2, b2.reshape(1, H),
        W3, b3.reshape(1, H),
    )


# speedup vs baseline: 1.0923x; 1.0754x over previous
import jax
import jax.numpy as jnp
from jax.experimental import pallas as pl

_BT = 8192


def _embed_kernel(x_ref, w0, b0, w1, b1, w2, b2, w3, b3, o_ref):
    ws = [w0, w1, w2, w3]
    bs = [b0, b1, b2, b3]
    wb = jnp.concatenate(
        [
            jnp.pad(ws[i][...], ((0, 0), (32 * i, 96 - 32 * i)))
            for i in range(4)
        ],
        axis=0,
    )
    bb = jnp.concatenate([b[...] for b in bs], axis=1)
    o_ref[...] = (
        jnp.dot(x_ref[...], wb, preferred_element_type=jnp.float32) + bb
    )


@jax.jit
def kernel(x, W0, b0, W1, b1, W2, b2, W3, b3):
    G, H = W0.shape
    D = G * 4
    O = H * 4
    B = x.shape[0]
    wspec = pl.BlockSpec((G, H), lambda i: (0, 0))
    bspec = pl.BlockSpec((1, H), lambda i: (0, 0))
    return pl.pallas_call(
        _embed_kernel,
        grid=(B // _BT,),
        in_specs=[
            pl.BlockSpec((_BT, D), lambda i: (i, 0)),
            wspec, bspec, wspec, bspec, wspec, bspec, wspec, bspec,
        ],
        out_specs=pl.BlockSpec((_BT, O), lambda i: (i, 0)),
        out_shape=jax.ShapeDtypeStruct((B, O), x.dtype),
    )(
        x,
        W0, b0.reshape(1, H),
        W1, b1.reshape(1, H),
        W2, b2.reshape(1, H),
        W3, b3.reshape(1, H),
    )
